# baseline (speedup 1.0000x reference)
import functools

import jax
import jax.numpy as jnp
from jax import lax
from jax.experimental import pallas as pl
from jax.experimental.pallas import tpu as pltpu
from jax.experimental.pallas import tpu_sc as plsc

_D = 256
_S = 512
_MAXLEN = 1023
_NW = 32
_RPW = _S // _NW

_mesh = plsc.VectorSubcoreMesh(core_axis_name="c", subcore_axis_name="s")


@functools.partial(
    pl.kernel,
    out_type=jax.ShapeDtypeStruct((_S, _S, _D), jnp.float32),
    mesh=_mesh,
    compiler_params=pltpu.CompilerParams(use_tc_tiling_on_sc=False),
    scratch_types=[
        pltpu.VMEM_SHARED((_MAXLEN, _D), jnp.float32),
        pltpu.SemaphoreType.DMA,
    ],
)
def _rpe_sc(flip_hbm, out_hbm, tab_spmem, sem):
    c = lax.axis_index("c")
    s = lax.axis_index("s")
    wid = s * 2 + c

    @pl.when(s < 15)
    def _stage():
        pltpu.sync_copy(flip_hbm.at[pl.ds(s * 64, 64), :],
                        tab_spmem.at[pl.ds(s * 64, 64), :])

    @pl.when(s == 15)
    def _stage_last():
        pltpu.sync_copy(flip_hbm.at[pl.ds(15 * 64, _MAXLEN - 15 * 64), :],
                        tab_spmem.at[pl.ds(15 * 64, _MAXLEN - 15 * 64), :])

    plsc.subcore_barrier()

    copies = []
    for r in range(_RPW):
        g = wid * _RPW + r
        start = (_S - 1) - g
        copies.append(
            pltpu.async_copy(tab_spmem.at[pl.ds(start, _S), :],
                             out_hbm.at[g], sem)
        )
    for cp in copies:
        cp.wait()


def kernel(x, rel_embeddings):
    flipped = rel_embeddings[::-1]
    rel_pos = _rpe_sc(flipped)
    return (x, rel_pos)


# speedup vs baseline: 2.5217x; 2.5217x over previous
import functools

import jax
import jax.numpy as jnp
from jax import lax
from jax.experimental import pallas as pl
from jax.experimental.pallas import tpu as pltpu
from jax.experimental.pallas import tpu_sc as plsc

_D = 256
_S = 512
_MAXLEN = 1023
_TAB = 1016
_NW = 32
_RPW = _S // _NW

_mesh = plsc.VectorSubcoreMesh(core_axis_name="c", subcore_axis_name="s")


def _build_shifted_kernel(flip_ref, a_ref):
    val = flip_ref[...]
    for k in range(8):
        rolled = val if k == 0 else pltpu.roll(val, 1024 - k, 0)
        a_ref[k] = rolled[0:_TAB]


@functools.partial(
    pl.kernel,
    out_type=jax.ShapeDtypeStruct((_S, _S, _D), jnp.float32),
    mesh=_mesh,
    scratch_types=[
        pltpu.VMEM_SHARED((8, _TAB, _D), jnp.float32),
        pltpu.SemaphoreType.DMA,
    ],
)
def _rpe_sc(a_hbm, out_hbm, tab_spmem, sem):
    c = lax.axis_index("c")
    s = lax.axis_index("s")
    wid = s * 2 + c

    @pl.when(s < 8)
    def _stage_lo():
        pltpu.sync_copy(a_hbm.at[s, pl.ds(0, 512), :],
                        tab_spmem.at[s, pl.ds(0, 512), :])

    @pl.when(s >= 8)
    def _stage_hi():
        pltpu.sync_copy(a_hbm.at[s - 8, pl.ds(512, _TAB - 512), :],
                        tab_spmem.at[s - 8, pl.ds(512, _TAB - 512), :])

    plsc.subcore_barrier()

    copies = []
    for r in range(_RPW):
        g = wid * _RPW + r
        k = (7 - r) % 8
        start = (_S - 1) - g
        q8 = pl.multiple_of(start - k, 8)
        copies.append(
            pltpu.async_copy(tab_spmem.at[k, pl.ds(q8, _S), :],
                             out_hbm.at[g], sem)
        )
    for cp in copies:
        cp.wait()


def kernel(x, rel_embeddings):
    flipped = rel_embeddings[::-1]
    flippad = jnp.pad(flipped, ((0, 1), (0, 0)))
    shifted = pl.pallas_call(
        _build_shifted_kernel,
        in_specs=[pl.BlockSpec((_MAXLEN + 1, _D), lambda: (0, 0))],
        out_specs=pl.BlockSpec((8, _TAB, _D), lambda: (0, 0, 0)),
        out_shape=jax.ShapeDtypeStruct((8, _TAB, _D), jnp.float32),
    )(flippad)
    rel_pos = _rpe_sc(shifted)
    return (x, rel_pos)


# speedup vs baseline: 2.6105x; 1.0352x over previous
import functools

import jax
import jax.numpy as jnp
from jax import lax
from jax.experimental import pallas as pl
from jax.experimental.pallas import tpu as pltpu
from jax.experimental.pallas import tpu_sc as plsc

_D = 256
_S = 512
_MAXLEN = 1023
_TAB = 1016
_NW = 32
_RPW = _S // _NW

_mesh = plsc.VectorSubcoreMesh(core_axis_name="c", subcore_axis_name="s")


def _build_shifted_kernel(tab_ref, a_ref):
    val = tab_ref[...]
    row = jax.lax.broadcasted_iota(jnp.int32, (_MAXLEN + 1, _D), 0)
    for b in range(10):
        stride = 1 << b
        hi = (row >> b) & 1 == 1
        val = jnp.where(hi, pltpu.roll(val, stride, 0),
                        pltpu.roll(val, 1024 - stride, 0))
    for k in range(8):
        rolled = pltpu.roll(val, 1024 - (k + 1), 0)
        a_ref[k] = rolled[0:_TAB]


@functools.partial(
    pl.kernel,
    out_type=jax.ShapeDtypeStruct((_S, _S, _D), jnp.float32),
    mesh=_mesh,
    scratch_types=[
        pltpu.VMEM_SHARED((8, _TAB, _D), jnp.float32),
        pltpu.SemaphoreType.DMA,
    ],
)
def _rpe_sc(a_hbm, out_hbm, tab_spmem, sem):
    c = lax.axis_index("c")
    s = lax.axis_index("s")
    wid = s * 2 + c

    @pl.when(s < 8)
    def _stage_lo():
        pltpu.sync_copy(a_hbm.at[s, pl.ds(0, 512), :],
                        tab_spmem.at[s, pl.ds(0, 512), :])

    @pl.when(s >= 8)
    def _stage_hi():
        pltpu.sync_copy(a_hbm.at[s - 8, pl.ds(512, _TAB - 512), :],
                        tab_spmem.at[s - 8, pl.ds(512, _TAB - 512), :])

    plsc.subcore_barrier()

    copies = []
    for r in range(_RPW):
        g = wid * _RPW + r
        k = (7 - r) % 8
        start = (_S - 1) - g
        q8 = pl.multiple_of(start - k, 8)
        copies.append(
            pltpu.async_copy(tab_spmem.at[k, pl.ds(q8, _S), :],
                             out_hbm.at[g], sem)
        )
    for cp in copies:
        cp.wait()


def kernel(x, rel_embeddings):
    shifted = pl.pallas_call(
        _build_shifted_kernel,
        grid=(1,),
        in_specs=[pl.BlockSpec((_MAXLEN + 1, _D), lambda i: (0, 0))],
        out_specs=pl.BlockSpec((8, _TAB, _D), lambda i: (0, 0, 0)),
        out_shape=jax.ShapeDtypeStruct((8, _TAB, _D), jnp.float32),
    )(rel_embeddings)
    rel_pos = _rpe_sc(shifted)
    return (x, rel_pos)
